# baseline (speedup 1.0000x reference)
import functools

import jax
import jax.numpy as jnp
from jax import lax
from jax.experimental import pallas as pl
from jax.experimental.pallas import tpu as pltpu
from jax.experimental.pallas import tpu_sc as plsc

_N = 10000
_E = 320000
_D = 128

_NC = 2
_NS = 16
_NW = _NC * _NS
_EPW = _E // _NW
_CH = 100
_NCH = _EPW // _CH
_SCH = 10
_NST = _NCH // _SCH
_NB = 3
_RPT = 624
_REM = _N - _NS * _RPT


def _segsum_body(h_hbm, idx_hbm, z_hbm, out_hbm, idx_v, rows0, rows1, rows2,
                 acc, gsem0, gsem1, gsem2, ssem0, ssem1, ssem2):
    cid = lax.axis_index("c")
    sid = lax.axis_index("s")
    wid = cid * _NS + sid
    rows = (rows0, rows1, rows2)
    gsem = (gsem0, gsem1, gsem2)
    ssem = (ssem0, ssem1, ssem2)

    base = sid * _RPT
    pltpu.sync_copy(z_hbm.at[pl.ds(base, _RPT)], acc.at[pl.ds(base, _RPT)])

    @pl.when(sid == _NS - 1)
    def _zrem():
        pltpu.sync_copy(z_hbm.at[pl.ds(_NS * _RPT, _REM)],
                        acc.at[pl.ds(_NS * _RPT, _REM)])

    plsc.subcore_barrier()

    def _stage(s, carry):
        pltpu.sync_copy(idx_hbm.at[wid].at[pl.ds(s * _SCH, _SCH)], idx_v)
        for b in range(_NB):
            pltpu.async_copy(h_hbm.at[idx_v.at[b, 0]], rows[b], gsem[b])
        for k in range(_SCH):
            b = k % _NB
            pltpu.make_async_copy(h_hbm.at[idx_v.at[k, 0]], rows[b],
                                  gsem[b]).wait()
            pltpu.async_copy(rows[b], acc.at[idx_v.at[k, 1]], ssem[b],
                             add=True)
            if k + _NB < _SCH:
                pltpu.make_async_copy(rows[b], acc.at[idx_v.at[k, 1]],
                                      ssem[b]).wait()
                pltpu.async_copy(h_hbm.at[idx_v.at[k + _NB, 0]], rows[b],
                                 gsem[b])
        for k in range(_SCH - _NB, _SCH):
            b = k % _NB
            pltpu.make_async_copy(rows[b], acc.at[idx_v.at[k, 1]],
                                  ssem[b]).wait()
        return carry

    lax.fori_loop(0, _NST, _stage, 0)
    plsc.subcore_barrier()

    pltpu.sync_copy(acc.at[pl.ds(base, _RPT)], out_hbm.at[cid, pl.ds(base, _RPT)])

    @pl.when(sid == _NS - 1)
    def _wrem():
        pltpu.sync_copy(acc.at[pl.ds(_NS * _RPT, _REM)],
                        out_hbm.at[cid, pl.ds(_NS * _RPT, _REM)])


@jax.jit
def _segsum_sc(h, idx, z):
    mesh = plsc.VectorSubcoreMesh(core_axis_name="c", subcore_axis_name="s")
    return pl.kernel(
        _segsum_body,
        out_type=jax.ShapeDtypeStruct((_NC, _N, _D), jnp.float32),
        mesh=mesh,
        scratch_types=[
            pltpu.VMEM((_SCH, 2, _CH), jnp.int32),
            pltpu.VMEM((_CH, _D), jnp.float32),
            pltpu.VMEM((_CH, _D), jnp.float32),
            pltpu.VMEM((_CH, _D), jnp.float32),
            pltpu.VMEM_SHARED((_N, _D), jnp.float32),
            pltpu.SemaphoreType.DMA,
            pltpu.SemaphoreType.DMA,
            pltpu.SemaphoreType.DMA,
            pltpu.SemaphoreType.DMA,
            pltpu.SemaphoreType.DMA,
            pltpu.SemaphoreType.DMA,
        ],
    )(h, idx, z)


def _mlp_body_bn(h_ref, a_ref, W1_ref, b1_ref, g1_ref, be1_ref, W2_ref, b2_ref,
                 bng_ref, bnb_ref, out_ref):
    _mlp_common(h_ref, a_ref, W1_ref, b1_ref, g1_ref, be1_ref, W2_ref, b2_ref,
                bng_ref, bnb_ref, out_ref)


def _mlp_body_nobn(h_ref, a_ref, W1_ref, b1_ref, g1_ref, be1_ref, W2_ref, b2_ref,
                   out_ref):
    _mlp_common(h_ref, a_ref, W1_ref, b1_ref, g1_ref, be1_ref, W2_ref, b2_ref,
                None, None, out_ref)


def _mlp_common(h_ref, a_ref, W1_ref, b1_ref, g1_ref, be1_ref, W2_ref, b2_ref,
                bng_ref, bnb_ref, out_ref):
    t = h_ref[...] + a_ref[0] + a_ref[1]
    u = jnp.dot(t, W1_ref[...], preferred_element_type=jnp.float32)
    u = u + b1_ref[...][None, :]
    m = jnp.mean(u, axis=0, keepdims=True)
    v = jnp.mean((u - m) * (u - m), axis=0, keepdims=True)
    u = (u - m) * lax.rsqrt(v + 1e-5) * g1_ref[...][None, :] + be1_ref[...][None, :]
    u = jnp.maximum(u, 0.0)
    h2 = jnp.dot(u, W2_ref[...], preferred_element_type=jnp.float32)
    h2 = h2 + b2_ref[...][None, :]
    if bng_ref is not None:
        m2 = jnp.mean(h2, axis=0, keepdims=True)
        v2 = jnp.mean((h2 - m2) * (h2 - m2), axis=0, keepdims=True)
        h2 = (h2 - m2) * lax.rsqrt(v2 + 1e-5) * bng_ref[...][None, :] \
            + bnb_ref[...][None, :]
    out_ref[...] = h2


def _mlp_tc(h, a, W1, b1, g1, be1, W2, b2, bng=None, bnb=None):
    out_shape = jax.ShapeDtypeStruct((_N, _D), jnp.float32)
    if bng is not None:
        return pl.pallas_call(_mlp_body_bn, out_shape=out_shape)(
            h, a, W1, b1, g1, be1, W2, b2, bng, bnb)
    return pl.pallas_call(_mlp_body_nobn, out_shape=out_shape)(
        h, a, W1, b1, g1, be1, W2, b2)


def kernel(x, g, W1_0, b1_0, g1_0, be1_0, W2_0, b2_0, W1_1, b1_1, g1_1, be1_1,
           W2_1, b2_1, W1_2, b1_2, g1_2, be1_2, W2_2, b2_2, bng_0, bnb_0,
           bng_1, bnb_1):
    src3 = g[0].reshape(_NW, _NCH, _CH)
    dst3 = g[1].reshape(_NW, _NCH, _CH)
    idx = jnp.stack([src3, dst3], axis=2)
    z = jnp.zeros((_N, _D), jnp.float32)

    a0 = _segsum_sc(x, idx, z)
    h1 = _mlp_tc(x, a0, W1_0, b1_0, g1_0, be1_0, W2_0, b2_0, bng_0, bnb_0)
    a1 = _segsum_sc(h1, idx, z)
    h2 = _mlp_tc(h1, a1, W1_1, b1_1, g1_1, be1_1, W2_1, b2_1, bng_1, bnb_1)
    a2 = _segsum_sc(h2, idx, z)
    return _mlp_tc(h2, a2, W1_2, b1_2, g1_2, be1_2, W2_2, b2_2)


# speedup vs baseline: 1.0747x; 1.0747x over previous
import functools

import jax
import jax.numpy as jnp
from jax import lax
from jax.experimental import pallas as pl
from jax.experimental.pallas import tpu as pltpu
from jax.experimental.pallas import tpu_sc as plsc

_N = 10000
_E = 320000
_D = 128

_NC = 2
_NS = 16
_NW = _NC * _NS
_EPW = _E // _NW
_CH = 125
_NCH = _EPW // _CH
_HCH = _NCH // 2
_RPT = 624
_REM = _N - _NS * _RPT


def _segsum_body(h_hbm, idx_hbm, z_hbm, out_hbm, idx_v, rows0, rows1,
                 acc, sem0, sem1):
    cid = lax.axis_index("c")
    sid = lax.axis_index("s")
    wid = cid * _NS + sid

    base = sid * _RPT
    pltpu.sync_copy(z_hbm.at[pl.ds(base, _RPT)], acc.at[pl.ds(base, _RPT)])

    @pl.when(sid == _NS - 1)
    def _zrem():
        pltpu.sync_copy(z_hbm.at[pl.ds(_NS * _RPT, _REM)],
                        acc.at[pl.ds(_NS * _RPT, _REM)])

    plsc.subcore_barrier()

    def _half(off):
        pltpu.sync_copy(idx_hbm.at[wid].at[pl.ds(off, _HCH)], idx_v)
        pltpu.async_copy(h_hbm.at[idx_v.at[0, 0]], rows0, sem0)
        pltpu.async_copy(h_hbm.at[idx_v.at[1, 0]], rows1, sem1)

        def _pair(i, carry):
            j0 = 2 * i
            pltpu.make_async_copy(h_hbm.at[idx_v.at[j0, 0]], rows0,
                                  sem0).wait()
            pltpu.sync_copy(rows0, acc.at[idx_v.at[j0, 1]], add=True)

            @pl.when(i + 1 < _HCH // 2)
            def _issue0():
                pltpu.async_copy(h_hbm.at[idx_v.at[j0 + 2, 0]], rows0, sem0)

            j1 = j0 + 1
            pltpu.make_async_copy(h_hbm.at[idx_v.at[j1, 0]], rows1,
                                  sem1).wait()
            pltpu.sync_copy(rows1, acc.at[idx_v.at[j1, 1]], add=True)

            @pl.when(i + 1 < _HCH // 2)
            def _issue1():
                pltpu.async_copy(h_hbm.at[idx_v.at[j1 + 2, 0]], rows1, sem1)
            return carry

        lax.fori_loop(0, _HCH // 2, _pair, 0)

    _half(0)
    _half(_HCH)
    plsc.subcore_barrier()

    pltpu.sync_copy(acc.at[pl.ds(base, _RPT)], out_hbm.at[cid, pl.ds(base, _RPT)])

    @pl.when(sid == _NS - 1)
    def _wrem():
        pltpu.sync_copy(acc.at[pl.ds(_NS * _RPT, _REM)],
                        out_hbm.at[cid, pl.ds(_NS * _RPT, _REM)])


@jax.jit
def _segsum_sc(h, idx, z):
    mesh = plsc.VectorSubcoreMesh(core_axis_name="c", subcore_axis_name="s")
    return pl.kernel(
        _segsum_body,
        out_type=jax.ShapeDtypeStruct((_NC, _N, _D), jnp.float32),
        mesh=mesh,
        scratch_types=[
            pltpu.VMEM((_HCH, 2, _CH), jnp.int32),
            pltpu.VMEM((_CH, _D), jnp.float32),
            pltpu.VMEM((_CH, _D), jnp.float32),
            pltpu.VMEM_SHARED((_N, _D), jnp.float32),
            pltpu.SemaphoreType.DMA,
            pltpu.SemaphoreType.DMA,
        ],
    )(h, idx, z)


def _mlp_body_bn(h_ref, a_ref, W1_ref, b1_ref, g1_ref, be1_ref, W2_ref, b2_ref,
                 bng_ref, bnb_ref, out_ref):
    _mlp_common(h_ref, a_ref, W1_ref, b1_ref, g1_ref, be1_ref, W2_ref, b2_ref,
                bng_ref, bnb_ref, out_ref)


def _mlp_body_nobn(h_ref, a_ref, W1_ref, b1_ref, g1_ref, be1_ref, W2_ref, b2_ref,
                   out_ref):
    _mlp_common(h_ref, a_ref, W1_ref, b1_ref, g1_ref, be1_ref, W2_ref, b2_ref,
                None, None, out_ref)


def _mlp_common(h_ref, a_ref, W1_ref, b1_ref, g1_ref, be1_ref, W2_ref, b2_ref,
                bng_ref, bnb_ref, out_ref):
    t = h_ref[...] + a_ref[0] + a_ref[1]
    u = jnp.dot(t, W1_ref[...], preferred_element_type=jnp.float32)
    u = u + b1_ref[...][None, :]
    m = jnp.mean(u, axis=0, keepdims=True)
    v = jnp.mean((u - m) * (u - m), axis=0, keepdims=True)
    u = (u - m) * lax.rsqrt(v + 1e-5) * g1_ref[...][None, :] + be1_ref[...][None, :]
    u = jnp.maximum(u, 0.0)
    h2 = jnp.dot(u, W2_ref[...], preferred_element_type=jnp.float32)
    h2 = h2 + b2_ref[...][None, :]
    if bng_ref is not None:
        m2 = jnp.mean(h2, axis=0, keepdims=True)
        v2 = jnp.mean((h2 - m2) * (h2 - m2), axis=0, keepdims=True)
        h2 = (h2 - m2) * lax.rsqrt(v2 + 1e-5) * bng_ref[...][None, :] \
            + bnb_ref[...][None, :]
    out_ref[...] = h2


def _mlp_tc(h, a, W1, b1, g1, be1, W2, b2, bng=None, bnb=None):
    out_shape = jax.ShapeDtypeStruct((_N, _D), jnp.float32)
    if bng is not None:
        return pl.pallas_call(_mlp_body_bn, out_shape=out_shape)(
            h, a, W1, b1, g1, be1, W2, b2, bng, bnb)
    return pl.pallas_call(_mlp_body_nobn, out_shape=out_shape)(
        h, a, W1, b1, g1, be1, W2, b2)


def kernel(x, g, W1_0, b1_0, g1_0, be1_0, W2_0, b2_0, W1_1, b1_1, g1_1, be1_1,
           W2_1, b2_1, W1_2, b1_2, g1_2, be1_2, W2_2, b2_2, bng_0, bnb_0,
           bng_1, bnb_1):
    src3 = g[0].reshape(_NW, _NCH, _CH)
    dst3 = g[1].reshape(_NW, _NCH, _CH)
    idx = jnp.stack([src3, dst3], axis=2)
    z = jnp.zeros((_N, _D), jnp.float32)

    a0 = _segsum_sc(x, idx, z)
    h1 = _mlp_tc(x, a0, W1_0, b1_0, g1_0, be1_0, W2_0, b2_0, bng_0, bnb_0)
    a1 = _segsum_sc(h1, idx, z)
    h2 = _mlp_tc(h1, a1, W1_1, b1_1, g1_1, be1_1, W2_1, b2_1, bng_1, bnb_1)
    a2 = _segsum_sc(h2, idx, z)
    return _mlp_tc(h2, a2, W1_2, b1_2, g1_2, be1_2, W2_2, b2_2)


# speedup vs baseline: 1.1255x; 1.0473x over previous
import functools

import jax
import jax.numpy as jnp
from jax import lax
from jax.experimental import pallas as pl
from jax.experimental.pallas import tpu as pltpu
from jax.experimental.pallas import tpu_sc as plsc

_N = 10000
_E = 320000
_D = 128

_NC = 2
_NS = 16
_NW = _NC * _NS
_EPW = _E // _NW
_CH = 125
_NCH = _EPW // _CH
_HCH = _NCH // 2
_RPT = 624
_REM = _N - _NS * _RPT
_ZROWS = 16


def _segsum_body(h_hbm, src_hbm, dst_hbm, out_hbm, src_v, dst_v, rows0, rows1,
                 zbuf, acc, sem0, sem1):
    cid = lax.axis_index("c")
    sid = lax.axis_index("s")
    wid = cid * _NS + sid

    zeros16 = jnp.zeros((16,), jnp.float32)

    def _zrow(r, carry):
        for j in range(_D // 16):
            zbuf[r, pl.ds(j * 16, 16)] = zeros16
        return carry

    lax.fori_loop(0, _ZROWS, _zrow, 0)
    base = sid * _RPT
    for i in range(_RPT // _ZROWS):
        pltpu.async_copy(zbuf, acc.at[pl.ds(base + i * _ZROWS, _ZROWS)], sem1)

    @pl.when(sid == _NS - 1)
    def _zrem():
        pltpu.async_copy(zbuf.at[pl.ds(0, _REM)],
                         acc.at[pl.ds(_NS * _RPT, _REM)], sem1)

    pltpu.sync_copy(src_hbm.at[wid].at[pl.ds(0, _HCH)], src_v)
    pltpu.sync_copy(dst_hbm.at[wid].at[pl.ds(0, _HCH)], dst_v)
    pltpu.async_copy(h_hbm.at[src_v.at[0]], rows0, sem0)

    for i in range(_RPT // _ZROWS):
        pltpu.make_async_copy(zbuf, acc.at[pl.ds(base + i * _ZROWS, _ZROWS)],
                              sem1).wait()

    @pl.when(sid == _NS - 1)
    def _zremw():
        pltpu.make_async_copy(zbuf.at[pl.ds(0, _REM)],
                              acc.at[pl.ds(_NS * _RPT, _REM)], sem1).wait()

    pltpu.async_copy(h_hbm.at[src_v.at[1]], rows1, sem1)
    plsc.subcore_barrier()

    def _half(off, stage):
        if stage:
            pltpu.sync_copy(src_hbm.at[wid].at[pl.ds(off, _HCH)], src_v)
            pltpu.sync_copy(dst_hbm.at[wid].at[pl.ds(off, _HCH)], dst_v)
            pltpu.async_copy(h_hbm.at[src_v.at[0]], rows0, sem0)
            pltpu.async_copy(h_hbm.at[src_v.at[1]], rows1, sem1)

        def _pair(i, carry):
            j0 = 2 * i
            pltpu.make_async_copy(h_hbm.at[src_v.at[j0]], rows0, sem0).wait()
            pltpu.sync_copy(rows0, acc.at[dst_v.at[j0]], add=True)

            @pl.when(i + 1 < _HCH // 2)
            def _issue0():
                pltpu.async_copy(h_hbm.at[src_v.at[j0 + 2]], rows0, sem0)

            j1 = j0 + 1
            pltpu.make_async_copy(h_hbm.at[src_v.at[j1]], rows1, sem1).wait()
            pltpu.sync_copy(rows1, acc.at[dst_v.at[j1]], add=True)

            @pl.when(i + 1 < _HCH // 2)
            def _issue1():
                pltpu.async_copy(h_hbm.at[src_v.at[j1 + 2]], rows1, sem1)
            return carry

        lax.fori_loop(0, _HCH // 2, _pair, 0)

    _half(0, stage=False)
    _half(_HCH, stage=True)
    plsc.subcore_barrier()

    pltpu.sync_copy(acc.at[pl.ds(base, _RPT)], out_hbm.at[cid, pl.ds(base, _RPT)])

    @pl.when(sid == _NS - 1)
    def _wrem():
        pltpu.sync_copy(acc.at[pl.ds(_NS * _RPT, _REM)],
                        out_hbm.at[cid, pl.ds(_NS * _RPT, _REM)])


@jax.jit
def _segsum_sc(h, src3, dst3):
    mesh = plsc.VectorSubcoreMesh(core_axis_name="c", subcore_axis_name="s")
    return pl.kernel(
        _segsum_body,
        out_type=jax.ShapeDtypeStruct((_NC, _N, _D), jnp.float32),
        mesh=mesh,
        scratch_types=[
            pltpu.VMEM((_HCH, _CH), jnp.int32),
            pltpu.VMEM((_HCH, _CH), jnp.int32),
            pltpu.VMEM((_CH, _D), jnp.float32),
            pltpu.VMEM((_CH, _D), jnp.float32),
            pltpu.VMEM((_ZROWS, _D), jnp.float32),
            pltpu.VMEM_SHARED((_N, _D), jnp.float32),
            pltpu.SemaphoreType.DMA,
            pltpu.SemaphoreType.DMA,
        ],
    )(h, src3, dst3)


def _mlp_body_bn(h_ref, a_ref, W1_ref, b1_ref, g1_ref, be1_ref, W2_ref, b2_ref,
                 bng_ref, bnb_ref, out_ref):
    _mlp_common(h_ref, a_ref, W1_ref, b1_ref, g1_ref, be1_ref, W2_ref, b2_ref,
                bng_ref, bnb_ref, out_ref)


def _mlp_body_nobn(h_ref, a_ref, W1_ref, b1_ref, g1_ref, be1_ref, W2_ref, b2_ref,
                   out_ref):
    _mlp_common(h_ref, a_ref, W1_ref, b1_ref, g1_ref, be1_ref, W2_ref, b2_ref,
                None, None, out_ref)


def _mlp_common(h_ref, a_ref, W1_ref, b1_ref, g1_ref, be1_ref, W2_ref, b2_ref,
                bng_ref, bnb_ref, out_ref):
    t = h_ref[...] + a_ref[0] + a_ref[1]
    u = jnp.dot(t, W1_ref[...], preferred_element_type=jnp.float32)
    u = u + b1_ref[...][None, :]
    m = jnp.mean(u, axis=0, keepdims=True)
    v = jnp.mean((u - m) * (u - m), axis=0, keepdims=True)
    u = (u - m) * lax.rsqrt(v + 1e-5) * g1_ref[...][None, :] + be1_ref[...][None, :]
    u = jnp.maximum(u, 0.0)
    h2 = jnp.dot(u, W2_ref[...], preferred_element_type=jnp.float32)
    h2 = h2 + b2_ref[...][None, :]
    if bng_ref is not None:
        m2 = jnp.mean(h2, axis=0, keepdims=True)
        v2 = jnp.mean((h2 - m2) * (h2 - m2), axis=0, keepdims=True)
        h2 = (h2 - m2) * lax.rsqrt(v2 + 1e-5) * bng_ref[...][None, :] \
            + bnb_ref[...][None, :]
    out_ref[...] = h2


def _mlp_tc(h, a, W1, b1, g1, be1, W2, b2, bng=None, bnb=None):
    out_shape = jax.ShapeDtypeStruct((_N, _D), jnp.float32)
    if bng is not None:
        return pl.pallas_call(_mlp_body_bn, out_shape=out_shape)(
            h, a, W1, b1, g1, be1, W2, b2, bng, bnb)
    return pl.pallas_call(_mlp_body_nobn, out_shape=out_shape)(
        h, a, W1, b1, g1, be1, W2, b2)


def kernel(x, g, W1_0, b1_0, g1_0, be1_0, W2_0, b2_0, W1_1, b1_1, g1_1, be1_1,
           W2_1, b2_1, W1_2, b1_2, g1_2, be1_2, W2_2, b2_2, bng_0, bnb_0,
           bng_1, bnb_1):
    src3 = g[0].reshape(_NW, _NCH, _CH)
    dst3 = g[1].reshape(_NW, _NCH, _CH)

    a0 = _segsum_sc(x, src3, dst3)
    h1 = _mlp_tc(x, a0, W1_0, b1_0, g1_0, be1_0, W2_0, b2_0, bng_0, bnb_0)
    a1 = _segsum_sc(h1, src3, dst3)
    h2 = _mlp_tc(h1, a1, W1_1, b1_1, g1_1, be1_1, W2_1, b2_1, bng_1, bnb_1)
    a2 = _segsum_sc(h2, src3, dst3)
    return _mlp_tc(h2, a2, W1_2, b1_2, g1_2, be1_2, W2_2, b2_2)
